# Initial kernel scaffold; baseline (speedup 1.0000x reference)
import jax
import jax.numpy as jnp
from jax.experimental import pallas as pl


def kernel(points, normals, pe_W1, pe_b1, pe_W2, pe_b2, lm_W1, lm_b1, lm_W2, lm_b2, gp_W, gp_b, cl_W1, cl_b1, cl_W2, cl_b2, ph_W1, ph_b1, ph_W2, ph_b2, bh_W1, bh_b1, bh_W2, bh_b2, ic_W, ic_b, ih_W1, ih_b1, ih_W2, ih_b2):
    raise NotImplementedError("write your pallas kernel here")


# speedup vs baseline: 17.2914x; 17.2914x over previous
import functools

import jax
import jax.numpy as jnp
from jax import lax
from jax.experimental import pallas as pl
from jax.experimental.pallas import tpu as pltpu
from jax.experimental.pallas import tpu_sc as plsc

B, N, H, G, C, P, I, K = 8, 2048, 128, 256, 16, 8, 64, 16
SCALE = 0.1
BN = B * N
T = 256
f32 = jnp.float32


def _dot(a, b):
    return jnp.dot(a, b, preferred_element_type=f32)


BLK_A = 2048


def _enc_body(x_ref, pw1_ref, pb1_ref, pw2_ref, pb2_ref, t_ref):
    x = x_ref[...]
    h = jnp.maximum(_dot(x, pw1_ref[...]) + pb1_ref[...], 0.0)
    f = jnp.maximum(_dot(h, pw2_ref[...]) + pb2_ref[...], 0.0)
    t_ref[:, 0:H] = f
    t_ref[:, H:H + 8] = x
    t_ref[:, H + 8:T] = jnp.zeros((BLK_A, T - H - 8), f32)


def _encoder(x8, pe_W1p, pe_b1, pe_W2, pe_b2):
    full = lambda shape: pl.BlockSpec(shape, lambda i: tuple(0 for _ in shape))
    return pl.pallas_call(
        _enc_body,
        grid=(BN // BLK_A,),
        in_specs=[
            pl.BlockSpec((BLK_A, 8), lambda i: (i, 0)),
            full((8, H)), full((1, H)), full((H, H)), full((1, H)),
        ],
        out_specs=pl.BlockSpec((BLK_A, T), lambda i: (i, 0)),
        out_shape=jax.ShapeDtypeStruct((BN, T), f32),
    )(x8, pe_W1p, pe_b1, pe_W2, pe_b2)


RB = 256


def _knn_body(p_ref, pt_ref, idx_ref):
    b = pl.program_id(0)
    p = p_ref[0]
    pt = pt_ref[0]
    xx_row = jnp.sum(p * p, axis=1, keepdims=True)
    xx_all = jnp.sum(pt * pt, axis=0, keepdims=True)
    d2 = (xx_row + xx_all) - 2.0 * _dot(p, pt)
    colid = lax.broadcasted_iota(jnp.int32, (RB, N), 1)
    base = b * N
    for t in range(K + 1):
        m = jnp.min(d2, axis=1, keepdims=True)
        am = jnp.min(jnp.where(d2 <= m, colid, N), axis=1, keepdims=True)
        d2 = jnp.where(colid == am, jnp.float32(jnp.inf), d2)
        if t > 0:
            idx_ref[0, :, pl.ds(t - 1, 1)] = am + base


def _knn(p4, p4t):
    return pl.pallas_call(
        _knn_body,
        grid=(B, N // RB),
        in_specs=[
            pl.BlockSpec((1, RB, 4), lambda b, j: (b, j, 0)),
            pl.BlockSpec((1, 4, N), lambda b, j: (b, 0, 0)),
        ],
        out_specs=pl.BlockSpec((1, RB, K), lambda b, j: (b, j, 0)),
        out_shape=jax.ShapeDtypeStruct((B, N, K), jnp.int32),
    )(p4, p4t)


_NW = 32
_CH = 128
_TOTAL = BN * K
_PER_W = _TOTAL // _NW
_NCH = _PER_W // _CH


def _sc_gather(table, idx_flat):
    mesh = plsc.VectorSubcoreMesh(core_axis_name="c", subcore_axis_name="s",
                                  num_cores=2, num_subcores=16)

    @functools.partial(
        pl.kernel,
        out_type=jax.ShapeDtypeStruct((_TOTAL, T), f32),
        mesh=mesh,
        scratch_types=[
            pltpu.VMEM((_CH,), jnp.int32),
            pltpu.VMEM((_CH, T), f32),
            pltpu.SemaphoreType.DMA,
        ],
    )
    def k(table_hbm, idx_hbm, out_hbm, idx_v, rows_v, sem):
        wid = lax.axis_index("s") * 2 + lax.axis_index("c")
        base = wid * _PER_W

        def body(c, carry):
            off = base + c * _CH
            pltpu.sync_copy(idx_hbm.at[pl.ds(off, _CH)], idx_v)
            pltpu.async_copy(table_hbm.at[idx_v], rows_v, sem).wait()
            pltpu.sync_copy(rows_v, out_hbm.at[pl.ds(off, _CH)])
            return carry

        lax.fori_loop(0, _NCH, body, 0)

    return k(table, idx_flat)


BD = 256


def _edge_body(g_ref, t_ref, w1a_ref, w1b_ref, w1pn_ref, b1_ref, w2_ref,
               b2_ref, loc_ref, gmx_ref, gsm_ref):
    j = pl.program_id(1)
    g = g_ref[0]
    t = t_ref[0]
    cf = t[:, 0:H]
    nf = g[:, :, 0:H]
    diff = (nf - cf[:, None, :]).reshape(BD * K, H)
    pdiff = (g[:, :, H:H + 8] - t[:, None, H:H + 8]).reshape(BD * K, 8)
    cterm = _dot(cf, w1a_ref[...]) + b1_ref[...]
    h1 = jnp.maximum(
        (_dot(diff, w1b_ref[...]) + _dot(pdiff, w1pn_ref[...])
         ).reshape(BD, K, H) + cterm[:, None, :], 0.0).reshape(BD * K, H)
    h2 = jnp.maximum(_dot(h1, w2_ref[...]) + b2_ref[...], 0.0).reshape(BD, K, H)
    loc = jnp.max(h2, axis=1)
    loc_ref[0] = loc
    bmx = jnp.max(loc, axis=0, keepdims=True)
    bsm = jnp.sum(loc, axis=0, keepdims=True)

    @pl.when(j == 0)
    def _():
        gmx_ref[0] = bmx
        gsm_ref[0] = bsm

    @pl.when(j > 0)
    def _():
        gmx_ref[0] = jnp.maximum(gmx_ref[0], bmx)
        gsm_ref[0] = gsm_ref[0] + bsm


def _edge(g, t3, w1a, w1b, w1pn, lm_b1, lm_W2, lm_b2):
    full = lambda shape: pl.BlockSpec(shape, lambda b, j: tuple(0 for _ in shape))
    return pl.pallas_call(
        _edge_body,
        grid=(B, N // BD),
        in_specs=[
            pl.BlockSpec((1, BD, K, T), lambda b, j: (b, j, 0, 0)),
            pl.BlockSpec((1, BD, T), lambda b, j: (b, j, 0)),
            full((H, H)), full((H, H)), full((8, H)), full((1, H)),
            full((H, H)), full((1, H)),
        ],
        out_specs=[
            pl.BlockSpec((1, BD, H), lambda b, j: (b, j, 0)),
            pl.BlockSpec((1, 1, H), lambda b, j: (b, 0, 0)),
            pl.BlockSpec((1, 1, H), lambda b, j: (b, 0, 0)),
        ],
        out_shape=[
            jax.ShapeDtypeStruct((B, N, H), f32),
            jax.ShapeDtypeStruct((B, 1, H), f32),
            jax.ShapeDtypeStruct((B, 1, H), f32),
        ],
    )(g, t3, w1a, w1b, w1pn, lm_b1, lm_W2, lm_b2)


BF = 1024
FD = 2 * H + G


def _heads_body(t_ref, l_ref, gmx_ref, gsm_ref, gpW_ref, gpb_ref, W_ref,
                bcat_ref, clW2_ref, clb2_ref, phW2_ref, phb2_ref, bhW2_ref,
                bhb2_ref, icW_ref, icb_ref, ihW1_ref, ihb1_ref, ihW2_ref,
                ihb2_ref, lg_ref, pr_ref, bd_ref, in_ref):
    f = t_ref[0][:, 0:H]
    l = l_ref[0]
    gcat = jnp.concatenate([gmx_ref[0], gsm_ref[0] * (1.0 / N)], axis=1)
    gfeat = jnp.maximum(_dot(gcat, gpW_ref[...]) + gpb_ref[...], 0.0)
    fused = jnp.concatenate(
        [f, l, jnp.broadcast_to(gfeat, (BF, G))], axis=1)
    h3 = _dot(fused, W_ref[...]) + bcat_ref[...]
    h_cl = jnp.maximum(h3[:, 0:H], 0.0)
    h_ph = jnp.maximum(h3[:, H:2 * H], 0.0)
    h_bh = jnp.maximum(h3[:, 2 * H:3 * H], 0.0)
    logits = _dot(h_cl, clW2_ref[...]) + clb2_ref[...]
    pr = _dot(h_ph, phW2_ref[...]) + phb2_ref[...]
    bd = _dot(h_bh, bhW2_ref[...]) + bhb2_ref[...]
    mx = jnp.max(logits, axis=1, keepdims=True)
    e = jnp.exp(logits - mx)
    sm = e / jnp.sum(e, axis=1, keepdims=True)
    ccv = _dot(sm, icW_ref[...]) + icb_ref[...]
    inst_in = fused + SCALE * ccv
    h_ih = jnp.maximum(_dot(inst_in, ihW1_ref[...]) + ihb1_ref[...], 0.0)
    inst = _dot(h_ih, ihW2_ref[...]) + ihb2_ref[...]
    nrm = jnp.sqrt(jnp.sum(inst * inst, axis=1, keepdims=True))
    inst = inst / jnp.maximum(nrm, 1e-12)
    lg_ref[0] = logits
    pr_ref[0] = pr
    bd_ref[0] = bd
    in_ref[0] = inst


def _heads(t3, loc3, gmx, gsm, gp_W, gp_b, Wcat, bcat, cl_W2, cl_b2, ph_W2,
           ph_b2, bh_W2, bh_b2, ic_W, ic_b, ih_W1, ih_b1, ih_W2, ih_b2):
    full = lambda shape: pl.BlockSpec(shape, lambda b, j: tuple(0 for _ in shape))
    return pl.pallas_call(
        _heads_body,
        grid=(B, N // BF),
        in_specs=[
            pl.BlockSpec((1, BF, T), lambda b, j: (b, j, 0)),
            pl.BlockSpec((1, BF, H), lambda b, j: (b, j, 0)),
            pl.BlockSpec((1, 1, H), lambda b, j: (b, 0, 0)),
            pl.BlockSpec((1, 1, H), lambda b, j: (b, 0, 0)),
            full((G, G)), full((1, G)), full((FD, 3 * H)), full((1, 3 * H)),
            full((H, C)), full((1, C)), full((H, P)), full((1, P)),
            full((H, 1)), full((1, 1)), full((C, FD)), full((1, FD)),
            full((FD, H)), full((1, H)), full((H, I)), full((1, I)),
        ],
        out_specs=[
            pl.BlockSpec((1, BF, C), lambda b, j: (b, j, 0)),
            pl.BlockSpec((1, BF, P), lambda b, j: (b, j, 0)),
            pl.BlockSpec((1, BF, 1), lambda b, j: (b, j, 0)),
            pl.BlockSpec((1, BF, I), lambda b, j: (b, j, 0)),
        ],
        out_shape=[
            jax.ShapeDtypeStruct((B, N, C), f32),
            jax.ShapeDtypeStruct((B, N, P), f32),
            jax.ShapeDtypeStruct((B, N, 1), f32),
            jax.ShapeDtypeStruct((B, N, I), f32),
        ],
    )(t3, loc3, gmx, gsm, gp_W, gp_b, Wcat, bcat, cl_W2, cl_b2, ph_W2, ph_b2,
      bh_W2, bh_b2, ic_W, ic_b, ih_W1, ih_b1, ih_W2, ih_b2)


def kernel(points, normals, pe_W1, pe_b1, pe_W2, pe_b2, lm_W1, lm_b1, lm_W2,
           lm_b2, gp_W, gp_b, cl_W1, cl_b1, cl_W2, cl_b2, ph_W1, ph_b1, ph_W2,
           ph_b2, bh_W1, bh_b1, bh_W2, bh_b2, ic_W, ic_b, ih_W1, ih_b1, ih_W2,
           ih_b2):
    x6 = jnp.concatenate([points, normals], axis=-1).reshape(BN, 6)
    x8 = jnp.pad(x6, ((0, 0), (0, 2)))
    pe_W1p = jnp.pad(pe_W1, ((0, 2), (0, 0)))
    row = lambda v: v.reshape(1, -1).astype(f32)

    table = _encoder(x8, pe_W1p, row(pe_b1), pe_W2, row(pe_b2))

    p4 = jnp.pad(points, ((0, 0), (0, 0), (0, 1)))
    p4t = jnp.transpose(p4, (0, 2, 1))
    idx = _knn(p4, p4t)

    g = _sc_gather(table, idx.reshape(_TOTAL))

    w1a = lm_W1[0:H]
    w1b = lm_W1[H:2 * H]
    w1pn = jnp.pad(lm_W1[2 * H:2 * H + 6], ((0, 2), (0, 0)))
    loc3, gmx, gsm = _edge(g.reshape(B, N, K, T), table.reshape(B, N, T),
                           w1a, w1b, w1pn, row(lm_b1), lm_W2, row(lm_b2))

    Wcat = jnp.concatenate([cl_W1, ph_W1, bh_W1], axis=1)
    bcat = jnp.concatenate([row(cl_b1), row(ph_b1), row(bh_b1)], axis=1)
    logits, pr, bd, inst = _heads(
        table.reshape(B, N, T), loc3, gmx, gsm, gp_W, row(gp_b), Wcat, bcat,
        cl_W2, row(cl_b2), ph_W2, row(ph_b2), bh_W2, row(bh_b2), ic_W,
        row(ic_b), ih_W1, row(ih_b1), ih_W2, row(ih_b2))
    return (logits, pr, bd.reshape(B, N), inst)


# speedup vs baseline: 22.6985x; 1.3127x over previous
import functools

import jax
import jax.numpy as jnp
from jax import lax
from jax.experimental import pallas as pl
from jax.experimental.pallas import tpu as pltpu
from jax.experimental.pallas import tpu_sc as plsc

B, N, H, G, C, P, I, K = 8, 2048, 128, 256, 16, 8, 64, 16
SCALE = 0.1
BN = B * N
T = H
f32 = jnp.float32


def _dot(a, b):
    return jnp.dot(a, b, preferred_element_type=f32)


BLK_A = 2048


def _enc_body(x_ref, pw1_ref, pb1_ref, pw2_ref, pb2_ref, t_ref):
    x = x_ref[...]
    h = jnp.maximum(_dot(x, pw1_ref[...]) + pb1_ref[...], 0.0)
    f = jnp.maximum(_dot(h, pw2_ref[...]) + pb2_ref[...], 0.0)
    t_ref[...] = f


def _encoder(x8, pe_W1p, pe_b1, pe_W2, pe_b2):
    full = lambda shape: pl.BlockSpec(shape, lambda i: tuple(0 for _ in shape))
    return pl.pallas_call(
        _enc_body,
        grid=(BN // BLK_A,),
        in_specs=[
            pl.BlockSpec((BLK_A, 8), lambda i: (i, 0)),
            full((8, H)), full((1, H)), full((H, H)), full((1, H)),
        ],
        out_specs=pl.BlockSpec((BLK_A, T), lambda i: (i, 0)),
        out_shape=jax.ShapeDtypeStruct((BN, T), f32),
    )(x8, pe_W1p, pe_b1, pe_W2, pe_b2)


RB = 256


def _knn_body(p_ref, pt_ref, idx_ref):
    b = pl.program_id(0)
    p = p_ref[0]
    pt = pt_ref[0]
    xx_row = jnp.sum(p * p, axis=1, keepdims=True)
    xx_all = jnp.sum(pt * pt, axis=0, keepdims=True)
    d2 = (xx_row + xx_all) - 2.0 * _dot(p, pt)
    colid = lax.broadcasted_iota(jnp.int32, (RB, N), 1) + b * N
    inf = jnp.float32(jnp.inf)
    big = jnp.int32(BN)
    for t in range(K + 1):
        m = jnp.min(d2, axis=1, keepdims=True)
        eq = d2 <= m
        if t > 0:
            am = jnp.min(jnp.where(eq, colid, big), axis=1, keepdims=True)
            idx_ref[0, :, pl.ds(t - 1, 1)] = am
        d2 = jnp.where(eq, inf, d2)


def _knn(p4, p4t):
    return pl.pallas_call(
        _knn_body,
        grid=(B, N // RB),
        in_specs=[
            pl.BlockSpec((1, RB, 4), lambda b, j: (b, j, 0)),
            pl.BlockSpec((1, 4, N), lambda b, j: (b, 0, 0)),
        ],
        out_specs=pl.BlockSpec((1, RB, K), lambda b, j: (b, j, 0)),
        out_shape=jax.ShapeDtypeStruct((B, N, K), jnp.int32),
    )(p4, p4t)


_NW = 32
_CH = 128
_TOTAL = BN * K
_PER_W = _TOTAL // _NW
_NCH = _PER_W // _CH


def _sc_gather(table, ctable, idx_flat):
    mesh = plsc.VectorSubcoreMesh(core_axis_name="c", subcore_axis_name="s",
                                  num_cores=2, num_subcores=16)

    @functools.partial(
        pl.kernel,
        out_type=(jax.ShapeDtypeStruct((_TOTAL, T), f32),
                  jax.ShapeDtypeStruct((_TOTAL, 8), f32)),
        mesh=mesh,
        scratch_types=[
            pltpu.VMEM((_CH,), jnp.int32), pltpu.VMEM((_CH,), jnp.int32),
            pltpu.VMEM((_CH, T), f32), pltpu.VMEM((_CH, T), f32),
            pltpu.VMEM((_CH, 8), f32), pltpu.VMEM((_CH, 8), f32),
            pltpu.SemaphoreType.DMA, pltpu.SemaphoreType.DMA,
        ],
        compiler_params=pltpu.CompilerParams(use_tc_tiling_on_sc=False),
    )
    def k(table_hbm, ctable_hbm, idx_hbm, out_hbm, outc_hbm,
          idx_v0, idx_v1, rows_v0, rows_v1, crows_v0, crows_v1, sem0, sem1):
        wid = lax.axis_index("s") * 2 + lax.axis_index("c")
        base = wid * _PER_W
        idx_v = (idx_v0, idx_v1)
        rows_v = (rows_v0, rows_v1)
        crows_v = (crows_v0, crows_v1)
        sem = (sem0, sem1)

        def start(c, bb):
            off = base + c * _CH
            pltpu.sync_copy(idx_hbm.at[pl.ds(off, _CH)], idx_v[bb])
            pltpu.make_async_copy(table_hbm.at[idx_v[bb]], rows_v[bb],
                                  sem[bb]).start()
            pltpu.make_async_copy(ctable_hbm.at[idx_v[bb]], crows_v[bb],
                                  sem[bb]).start()

        def drain(c, bb):
            off = base + c * _CH
            pltpu.make_async_copy(table_hbm.at[idx_v[bb]], rows_v[bb],
                                  sem[bb]).wait()
            pltpu.make_async_copy(ctable_hbm.at[idx_v[bb]], crows_v[bb],
                                  sem[bb]).wait()
            pltpu.sync_copy(rows_v[bb], out_hbm.at[pl.ds(off, _CH)])
            pltpu.sync_copy(crows_v[bb], outc_hbm.at[pl.ds(off, _CH)])

        start(0, 0)
        start(1, 1)

        def body(pi, carry):
            for bb in range(2):
                i = pi * 2 + bb
                drain(i, bb)

                @pl.when(i + 2 < _NCH)
                def _():
                    start(i + 2, bb)

            return carry

        lax.fori_loop(0, _NCH // 2, body, 0)

    return k(table, ctable, idx_flat)


BD = 256


def _edge_body(g_ref, pn_ref, t_ref, x_ref, w1a_ref, w1b_ref, wbd_ref,
               b1_ref, w2_ref, b2_ref, loc_ref, gmx_ref, gsm_ref):
    j = pl.program_id(1)
    g = g_ref[0]
    cf = t_ref[0]
    pn = pn_ref[0]
    x = x_ref[0]
    diff = (g - cf[:, None, :]).reshape(BD * K, H)
    pd = pn - jnp.concatenate([x] * K, axis=1)
    hpn = _dot(pd, wbd_ref[...]).reshape(BD, K, H)
    cterm = _dot(cf, w1a_ref[...]) + b1_ref[...]
    h1 = jnp.maximum(
        _dot(diff, w1b_ref[...]).reshape(BD, K, H) + hpn
        + cterm[:, None, :], 0.0).reshape(BD * K, H)
    h2 = jnp.maximum(_dot(h1, w2_ref[...]) + b2_ref[...], 0.0).reshape(BD, K, H)
    loc = jnp.max(h2, axis=1)
    loc_ref[0] = loc
    bmx = jnp.max(loc, axis=0, keepdims=True)
    bsm = jnp.sum(loc, axis=0, keepdims=True)

    @pl.when(j == 0)
    def _():
        gmx_ref[0] = bmx
        gsm_ref[0] = bsm

    @pl.when(j > 0)
    def _():
        gmx_ref[0] = jnp.maximum(gmx_ref[0], bmx)
        gsm_ref[0] = gsm_ref[0] + bsm


def _edge(g, pn3, t3, x83, w1a, w1b, wbd, lm_b1, lm_W2, lm_b2):
    full = lambda shape: pl.BlockSpec(shape, lambda b, j: tuple(0 for _ in shape))
    return pl.pallas_call(
        _edge_body,
        grid=(B, N // BD),
        in_specs=[
            pl.BlockSpec((1, BD, K, H), lambda b, j: (b, j, 0, 0)),
            pl.BlockSpec((1, BD, K * 8), lambda b, j: (b, j, 0)),
            pl.BlockSpec((1, BD, H), lambda b, j: (b, j, 0)),
            pl.BlockSpec((1, BD, 8), lambda b, j: (b, j, 0)),
            full((H, H)), full((H, H)), full((K * 8, K * H)), full((1, H)),
            full((H, H)), full((1, H)),
        ],
        out_specs=[
            pl.BlockSpec((1, BD, H), lambda b, j: (b, j, 0)),
            pl.BlockSpec((1, 1, H), lambda b, j: (b, 0, 0)),
            pl.BlockSpec((1, 1, H), lambda b, j: (b, 0, 0)),
        ],
        out_shape=[
            jax.ShapeDtypeStruct((B, N, H), f32),
            jax.ShapeDtypeStruct((B, 1, H), f32),
            jax.ShapeDtypeStruct((B, 1, H), f32),
        ],
    )(g, pn3, t3, x83, w1a, w1b, wbd, lm_b1, lm_W2, lm_b2)


BF = 1024
FD = 2 * H + G


def _heads_body(t_ref, l_ref, gmx_ref, gsm_ref, gpW_ref, gpb_ref, W_ref,
                bcat_ref, clW2_ref, clb2_ref, phW2_ref, phb2_ref, bhW2_ref,
                bhb2_ref, icW_ref, icb_ref, ihW1_ref, ihb1_ref, ihW2_ref,
                ihb2_ref, lg_ref, pr_ref, bd_ref, in_ref):
    f = t_ref[0][:, 0:H]
    l = l_ref[0]
    gcat = jnp.concatenate([gmx_ref[0], gsm_ref[0] * (1.0 / N)], axis=1)
    gfeat = jnp.maximum(_dot(gcat, gpW_ref[...]) + gpb_ref[...], 0.0)
    fused = jnp.concatenate(
        [f, l, jnp.broadcast_to(gfeat, (BF, G))], axis=1)
    h3 = _dot(fused, W_ref[...]) + bcat_ref[...]
    h_cl = jnp.maximum(h3[:, 0:H], 0.0)
    h_ph = jnp.maximum(h3[:, H:2 * H], 0.0)
    h_bh = jnp.maximum(h3[:, 2 * H:3 * H], 0.0)
    logits = _dot(h_cl, clW2_ref[...]) + clb2_ref[...]
    pr = _dot(h_ph, phW2_ref[...]) + phb2_ref[...]
    bd = _dot(h_bh, bhW2_ref[...]) + bhb2_ref[...]
    mx = jnp.max(logits, axis=1, keepdims=True)
    e = jnp.exp(logits - mx)
    sm = e / jnp.sum(e, axis=1, keepdims=True)
    ccv = _dot(sm, icW_ref[...]) + icb_ref[...]
    inst_in = fused + SCALE * ccv
    h_ih = jnp.maximum(_dot(inst_in, ihW1_ref[...]) + ihb1_ref[...], 0.0)
    inst = _dot(h_ih, ihW2_ref[...]) + ihb2_ref[...]
    nrm = jnp.sqrt(jnp.sum(inst * inst, axis=1, keepdims=True))
    inst = inst / jnp.maximum(nrm, 1e-12)
    lg_ref[0] = logits
    pr_ref[0] = pr
    bd_ref[0] = bd
    in_ref[0] = inst


def _heads(t3, loc3, gmx, gsm, gp_W, gp_b, Wcat, bcat, cl_W2, cl_b2, ph_W2,
           ph_b2, bh_W2, bh_b2, ic_W, ic_b, ih_W1, ih_b1, ih_W2, ih_b2):
    full = lambda shape: pl.BlockSpec(shape, lambda b, j: tuple(0 for _ in shape))
    return pl.pallas_call(
        _heads_body,
        grid=(B, N // BF),
        in_specs=[
            pl.BlockSpec((1, BF, T), lambda b, j: (b, j, 0)),
            pl.BlockSpec((1, BF, H), lambda b, j: (b, j, 0)),
            pl.BlockSpec((1, 1, H), lambda b, j: (b, 0, 0)),
            pl.BlockSpec((1, 1, H), lambda b, j: (b, 0, 0)),
            full((G, G)), full((1, G)), full((FD, 3 * H)), full((1, 3 * H)),
            full((H, C)), full((1, C)), full((H, P)), full((1, P)),
            full((H, 1)), full((1, 1)), full((C, FD)), full((1, FD)),
            full((FD, H)), full((1, H)), full((H, I)), full((1, I)),
        ],
        out_specs=[
            pl.BlockSpec((1, BF, C), lambda b, j: (b, j, 0)),
            pl.BlockSpec((1, BF, P), lambda b, j: (b, j, 0)),
            pl.BlockSpec((1, BF, 1), lambda b, j: (b, j, 0)),
            pl.BlockSpec((1, BF, I), lambda b, j: (b, j, 0)),
        ],
        out_shape=[
            jax.ShapeDtypeStruct((B, N, C), f32),
            jax.ShapeDtypeStruct((B, N, P), f32),
            jax.ShapeDtypeStruct((B, N, 1), f32),
            jax.ShapeDtypeStruct((B, N, I), f32),
        ],
    )(t3, loc3, gmx, gsm, gp_W, gp_b, Wcat, bcat, cl_W2, cl_b2, ph_W2, ph_b2,
      bh_W2, bh_b2, ic_W, ic_b, ih_W1, ih_b1, ih_W2, ih_b2)


def kernel(points, normals, pe_W1, pe_b1, pe_W2, pe_b2, lm_W1, lm_b1, lm_W2,
           lm_b2, gp_W, gp_b, cl_W1, cl_b1, cl_W2, cl_b2, ph_W1, ph_b1, ph_W2,
           ph_b2, bh_W1, bh_b1, bh_W2, bh_b2, ic_W, ic_b, ih_W1, ih_b1, ih_W2,
           ih_b2):
    x6 = jnp.concatenate([points, normals], axis=-1).reshape(BN, 6)
    x8 = jnp.pad(x6, ((0, 0), (0, 2)))
    pe_W1p = jnp.pad(pe_W1, ((0, 2), (0, 0)))
    row = lambda v: v.reshape(1, -1).astype(f32)

    table = _encoder(x8, pe_W1p, row(pe_b1), pe_W2, row(pe_b2))

    p4 = jnp.pad(points, ((0, 0), (0, 0), (0, 1)))
    p4t = jnp.transpose(p4, (0, 2, 1))
    idx = _knn(p4, p4t)

    g, gc = _sc_gather(table, x8, idx.reshape(_TOTAL))

    w1a = lm_W1[0:H]
    w1b = lm_W1[H:2 * H]
    w1pn = jnp.pad(lm_W1[2 * H:2 * H + 6], ((0, 2), (0, 0)))
    wbd = jnp.kron(jnp.eye(K, dtype=f32), w1pn)
    loc3, gmx, gsm = _edge(g.reshape(B, N, K, H), gc.reshape(B, N, K * 8),
                           table.reshape(B, N, H), x8.reshape(B, N, 8),
                           w1a, w1b, wbd, row(lm_b1), lm_W2, row(lm_b2))

    Wcat = jnp.concatenate([cl_W1, ph_W1, bh_W1], axis=1)
    bcat = jnp.concatenate([row(cl_b1), row(ph_b1), row(bh_b1)], axis=1)
    logits, pr, bd, inst = _heads(
        table.reshape(B, N, T), loc3, gmx, gsm, gp_W, row(gp_b), Wcat, bcat,
        cl_W2, row(cl_b2), ph_W2, row(ph_b2), bh_W2, row(bh_b2), ic_W,
        row(ic_b), ih_W1, row(ih_b1), ih_W2, row(ih_b2))
    return (logits, pr, bd.reshape(B, N), inst)
